# baseline (speedup 1.0000x reference)
import functools

import jax
import jax.numpy as jnp
from jax import lax
from jax.experimental import pallas as pl
from jax.experimental.pallas import tpu as pltpu
from jax.experimental.pallas import tpu_sc as plsc

B = 16384
NS = 26
ND = 13
D = 32
NF = NS + ND

NC = 2
NSUB = 16
NW = NC * NSUB

SP_TOT = B * NS
BW = B // NW
CB = 32
CH = CB * NS
NQ = BW // CB

VOCAB = 1000000
CHV = 8192
QV = CHV // 4
VGRID = (VOCAB + CHV - 1) // CHV
VPAD = VGRID * CHV

FT = NF * 4
FTS = NS * 4


def _pack_body(xt_ref, o_ref):
    x = xt_ref[...]
    acc = None
    for a in range(4):
        xa = x[:, a * QV:(a + 1) * QV]
        rr = jax.lax.broadcasted_iota(jnp.int32, (D, 128), 0)
        cc = jax.lax.broadcasted_iota(jnp.int32, (D, 128), 1)
        ea = (cc == rr + a * D).astype(jnp.float32)
        ta = jax.lax.dot_general(xa, ea, (((0,), (0,)), ((), ())),
                                 preferred_element_type=jnp.float32)
        acc = ta if acc is None else acc + ta
    o_ref[...] = acc


def _pack_table(emb_table):
    packed = pl.pallas_call(
        _pack_body,
        grid=(VGRID,),
        in_specs=[pl.BlockSpec((D, CHV), lambda i: (0, i))],
        out_specs=pl.BlockSpec((CHV // 4, 128), lambda i: (i, 0)),
        out_shape=jax.ShapeDtypeStruct((VPAD // 4, 128), jnp.float32),
    )(emb_table.T)
    return packed.reshape(VPAD, D)


DGC = 8


def _dense_body(xt_ref, wt_ref, _, o_ref):
    w = wt_ref[...]
    for j in range(DGC):
        prod = jax.lax.dot_general(
            w, xt_ref[:, j * 128:(j + 1) * 128], (((1,), (0,)), ((), ())),
            preferred_element_type=jnp.float32)
        o_ref[:, j, :, :] = prod.reshape(ND * 4, 8, 128)


def _dense_into(dense_inputs, w_dense, sc_out):
    return pl.pallas_call(
        _dense_body,
        grid=(128 // DGC,),
        in_specs=[
            pl.BlockSpec((ND, DGC * 128), lambda i: (0, i)),
            pl.BlockSpec((ND * D, ND), lambda i: (0, 0)),
            pl.BlockSpec(memory_space=pltpu.MemorySpace.HBM),
        ],
        out_specs=pl.BlockSpec((ND * 4, DGC, 8, 128), lambda i: (2, i, 0, 0)),
        out_shape=jax.ShapeDtypeStruct((FT, 128, 8, 128), jnp.float32),
        input_output_aliases={2: 0},
    )(dense_inputs.T, w_dense.T, sc_out)


_mesh = plsc.VectorSubcoreMesh(core_axis_name="c", subcore_axis_name="s")


@functools.partial(
    pl.kernel,
    out_type=jax.ShapeDtypeStruct((FT, 128, 8, 128), jnp.float32),
    mesh=_mesh,
    scratch_types=(
        [pltpu.VMEM((BW * NS,), jnp.int32)]
        + [pltpu.VMEM((CH, D), jnp.float32) for _ in range(2)]
        + [pltpu.VMEM((FTS, 8, CB), jnp.float32) for _ in range(2)]
        + [pltpu.SemaphoreType.DMA for _ in range(5)]
    ),
    compiler_params=pltpu.CompilerParams(use_tc_tiling_on_sc=False,
                                         needs_layout_passes=False),
)
def _sc_sparse(idx_hbm, table_hbm, out_hbm,
               idx_all, rows0, rows1, asm0, asm1,
               isem, gsem0, gsem1, wsem0, wsem1):
    rows = (rows0, rows1)
    asms = (asm0, asm1)
    gsems = (gsem0, gsem1)
    wsems = (wsem0, wsem1)

    wid = lax.axis_index("s") * NC + lax.axis_index("c")
    c_base = wid * 4

    ii = jnp.arange(16, dtype=jnp.int32) * NS

    def gather(q, b):
        src = table_hbm.at[idx_all.at[pl.ds(q * CH, CH)]]
        return pltpu.async_copy(src, rows[b], gsems[b])

    def write(q, b):
        c_abs = c_base + q // 4
        l0 = (q % 4) * CB
        dst = out_hbm.at[pl.ds(0, FTS), c_abs, :, pl.ds(l0, CB)]
        return pltpu.async_copy(asms[b], dst, wsems[b])

    def transpose(b):
        rbuf = rows[b]
        abuf = asms[b]

        @plsc.parallel_loop(0, NS, unroll=4)
        def fbody(f):
            for g in range(CB // 16):
                ridx = ii + (g * 16 * NS + f)
                for t in range(4):
                    for s in range(8):
                        cidx = jnp.full((16,), 8 * t + s, jnp.int32)
                        vec = plsc.load_gather(rbuf, [ridx, cidx])
                        abuf[4 * f + t, s, pl.ds(g * 16, 16)] = vec

    pltpu.sync_copy(idx_hbm.at[pl.ds(wid * BW * NS, BW * NS)], idx_all)
    gather(0, 0)

    def pair(p, _):
        for a in range(2):
            b = a
            q = 2 * p + a
            nxt = 1 - b
            if a == 0:
                pltpu.make_async_copy(
                    table_hbm.at[idx_all.at[pl.ds(0, CH)]], rows[b], gsems[b]
                ).wait()
                gather(q + 1, nxt)
            else:
                @pl.when(p < NQ // 2 - 1)
                def _():
                    gather(q + 1, nxt)
                pltpu.make_async_copy(
                    table_hbm.at[idx_all.at[pl.ds(0, CH)]], rows[b], gsems[b]
                ).wait()

            @pl.when(p > 0)
            def _():
                dst = out_hbm.at[pl.ds(0, FTS), 0, :, pl.ds(0, CB)]
                pltpu.make_async_copy(asms[b], dst, wsems[b]).wait()

            transpose(b)
            write(q, b)
        return 0

    lax.fori_loop(0, NQ // 2, pair, 0)
    for b in range(2):
        dst = out_hbm.at[pl.ds(0, FTS), 0, :, pl.ds(0, CB)]
        pltpu.make_async_copy(asms[b], dst, wsems[b]).wait()


def kernel(sparse_inputs, dense_inputs, emb_table, W_dense):
    v = sparse_inputs.astype(jnp.int32).reshape(SP_TOT)
    flat_idx = ((v & ~(CHV - 1)) | ((v & (QV - 1)) << 2)
                | ((v & (CHV - 1)) >> (QV.bit_length() - 1)))
    sc_out = _sc_sparse(flat_idx, _pack_table(emb_table))
    full = _dense_into(dense_inputs, W_dense, sc_out)
    x5 = full.reshape(NF, 4, 128, 8, 128)
    return x5.transpose(2, 4, 0, 1, 3).reshape(B, NF, D)


# speedup vs baseline: 1.1375x; 1.1375x over previous
import functools

import jax
import jax.numpy as jnp
from jax import lax
from jax.experimental import pallas as pl
from jax.experimental.pallas import tpu as pltpu
from jax.experimental.pallas import tpu_sc as plsc

B = 16384
NS = 26
ND = 13
D = 32
NF = NS + ND

NC = 2
NSUB = 16
NW = NC * NSUB

SP_TOT = B * NS
BW = B // NW
CB = 32
CH = CB * NS
NQ = BW // CB

VOCAB = 1000000
CHV = 8192
QV = CHV // 4
VGRID = (VOCAB + CHV - 1) // CHV
VPAD = VGRID * CHV

FT = NF * 4
FTS = NS * 4


def _pack_body(xt_ref, o_ref):
    x = xt_ref[...]
    acc = None
    for a in range(4):
        xa = x[:, a * QV:(a + 1) * QV]
        rr = jax.lax.broadcasted_iota(jnp.int32, (D, 128), 0)
        cc = jax.lax.broadcasted_iota(jnp.int32, (D, 128), 1)
        ea = (cc == rr + a * D).astype(jnp.float32)
        ta = jax.lax.dot_general(xa, ea, (((0,), (0,)), ((), ())),
                                 preferred_element_type=jnp.float32)
        acc = ta if acc is None else acc + ta
    o_ref[...] = acc


def _pack_table(emb_table):
    packed = pl.pallas_call(
        _pack_body,
        grid=(VGRID,),
        in_specs=[pl.BlockSpec((D, CHV), lambda i: (0, i))],
        out_specs=pl.BlockSpec((CHV // 4, 128), lambda i: (i, 0)),
        out_shape=jax.ShapeDtypeStruct((VPAD // 4, 128), jnp.float32),
    )(emb_table.T)
    return packed.reshape(VPAD, D)


DGC = 8


def _dense_body(xt_ref, wt_ref, _, o_ref):
    w = wt_ref[...]
    for j in range(DGC):
        prod = jax.lax.dot_general(
            w, xt_ref[:, j * 128:(j + 1) * 128], (((1,), (0,)), ((), ())),
            preferred_element_type=jnp.float32)
        o_ref[:, j, :, :] = prod.reshape(ND * 4, 8, 128)


def _dense_into(dense_inputs, w_dense, sc_out):
    return pl.pallas_call(
        _dense_body,
        grid=(128 // DGC,),
        in_specs=[
            pl.BlockSpec((ND, DGC * 128), lambda i: (0, i)),
            pl.BlockSpec((ND * D, ND), lambda i: (0, 0)),
            pl.BlockSpec(memory_space=pltpu.MemorySpace.HBM),
        ],
        out_specs=pl.BlockSpec((ND * 4, DGC, 8, 128), lambda i: (2, i, 0, 0)),
        out_shape=jax.ShapeDtypeStruct((FT, 128, 8, 128), jnp.float32),
        input_output_aliases={2: 0},
    )(dense_inputs.T, w_dense.T, sc_out)


_mesh = plsc.VectorSubcoreMesh(core_axis_name="c", subcore_axis_name="s")


@functools.partial(
    pl.kernel,
    out_type=jax.ShapeDtypeStruct((FT, 128, 8, 128), jnp.float32),
    mesh=_mesh,
    scratch_types=(
        [pltpu.VMEM((BW * NS,), jnp.int32)]
        + [pltpu.VMEM((CH, D), jnp.float32) for _ in range(2)]
        + [pltpu.VMEM((FTS, 8, CB), jnp.float32) for _ in range(2)]
        + [pltpu.SemaphoreType.DMA for _ in range(5)]
    ),
    compiler_params=pltpu.CompilerParams(use_tc_tiling_on_sc=False,
                                         needs_layout_passes=False),
)
def _sc_sparse(idx_hbm, table_hbm, out_hbm,
               idx_all, rows0, rows1, asm0, asm1,
               isem, gsem0, gsem1, wsem0, wsem1):
    rows = (rows0, rows1)
    asms = (asm0, asm1)
    gsems = (gsem0, gsem1)
    wsems = (wsem0, wsem1)

    wid = lax.axis_index("s") * NC + lax.axis_index("c")
    c_base = wid * 4

    ii = jnp.arange(16, dtype=jnp.int32) * NS

    def gather(q, b):
        src = table_hbm.at[idx_all.at[pl.ds(q * CH, CH)]]
        return pltpu.async_copy(src, rows[b], gsems[b])

    def write(q, b):
        c_abs = c_base + q // 4
        l0 = (q % 4) * CB
        dst = out_hbm.at[pl.ds(0, FTS), c_abs, :, pl.ds(l0, CB)]
        return pltpu.async_copy(asms[b], dst, wsems[b])

    def transpose(b):
        rbuf = rows[b]
        abuf = asms[b]

        @plsc.parallel_loop(0, NS, unroll=2)
        def fbody(f):
            for g in range(CB // 16):
                ridx = ii + (g * 16 * NS + f)
                for t in range(4):
                    for s in range(8):
                        cidx = jnp.full((16,), 8 * t + s, jnp.int32)
                        vec = plsc.load_gather(rbuf, [ridx, cidx])
                        abuf[4 * f + t, s, pl.ds(g * 16, 16)] = vec

    pltpu.sync_copy(idx_hbm.at[pl.ds(wid * BW * NS, BW * NS)], idx_all)
    gather(0, 0)

    def pair(p, _):
        for a in range(2):
            b = a
            q = 2 * p + a
            nxt = 1 - b
            if a == 0:
                pltpu.make_async_copy(
                    table_hbm.at[idx_all.at[pl.ds(0, CH)]], rows[b], gsems[b]
                ).wait()
                gather(q + 1, nxt)
            else:
                @pl.when(p < NQ // 2 - 1)
                def _():
                    gather(q + 1, nxt)
                pltpu.make_async_copy(
                    table_hbm.at[idx_all.at[pl.ds(0, CH)]], rows[b], gsems[b]
                ).wait()

            @pl.when(p > 0)
            def _():
                dst = out_hbm.at[pl.ds(0, FTS), 0, :, pl.ds(0, CB)]
                pltpu.make_async_copy(asms[b], dst, wsems[b]).wait()

            transpose(b)
            write(q, b)
        return 0

    lax.fori_loop(0, NQ // 2, pair, 0)
    for b in range(2):
        dst = out_hbm.at[pl.ds(0, FTS), 0, :, pl.ds(0, CB)]
        pltpu.make_async_copy(asms[b], dst, wsems[b]).wait()


def kernel(sparse_inputs, dense_inputs, emb_table, W_dense):
    v = sparse_inputs.astype(jnp.int32).reshape(SP_TOT)
    flat_idx = ((v & ~(CHV - 1)) | ((v & (QV - 1)) << 2)
                | ((v & (CHV - 1)) >> (QV.bit_length() - 1)))
    sc_out = _sc_sparse(flat_idx, _pack_table(emb_table))
    full = _dense_into(dense_inputs, W_dense, sc_out)
    x5 = full.reshape(NF, 4, 128, 8, 128)
    return x5.transpose(2, 4, 0, 1, 3).reshape(B, NF, D)
